# baseline (speedup 1.0000x reference)
import jax
import jax.numpy as jnp
from jax import lax
from jax.experimental import pallas as pl
from jax.experimental.pallas import tpu as pltpu
from jax.experimental.pallas import tpu_sc as plsc

B, C, H, W = 1, 32, 512, 512
HT_H, HT_W = 400, 400
N_BINS = HT_H * HT_W
N_VOTES = 2000000
NC, NS, L = 2, 16, 16
N_CHUNKS = 4
CHUNK = N_BINS // N_CHUNKS
ACC_ROWS = CHUNK + 160
BLK = 128
PADN = 1 << 21
VROWS = PADN // BLK
TROWS = VROWS // NS
SUP = 32
STEPS = TROWS // SUP
RING = 3 * BLK
RSLACK = RING + BLK
STRIPE = 2560
WB_LAST = CHUNK - (NS - 1) * STRIPE
Z_LAST = ACC_ROWS - (NS - 1) * STRIPE
ZROWS = 160


def _sc_body(x_hbm, im_hbm, ht_hbm, w_hbm, out_hbm,
             acc, ist, hst, wstg, r_im, r_ht, r_w, sht2, swt2, rows2, zbuf,
             sem_st, sem_g0, sem_g1, sem_s0, sem_s1):
  cid = lax.axis_index("c")
  sid = lax.axis_index("s")

  def _zfill(i, c):
    zbuf[i, pl.ds(0, L)] = jnp.zeros((L,), jnp.float32)
    zbuf[i, pl.ds(L, L)] = jnp.zeros((L,), jnp.float32)
    return c
  lax.fori_loop(0, ZROWS, _zfill, 0)
  def _rfill(i, c):
    r_im[pl.ds(i * L, L)] = jnp.zeros((L,), jnp.int32)
    r_ht[pl.ds(i * L, L)] = jnp.full((L,), CHUNK, jnp.int32)
    r_w[pl.ds(i * L, L)] = jnp.zeros((L,), jnp.float32)
    return c
  lax.fori_loop(0, RSLACK // L, _rfill, 0)

  for chunk_i in range(N_CHUNKS // NC):
    lo = (cid * (N_CHUNKS // NC) + chunk_i) * CHUNK

    zstart = sid * STRIPE
    nz = jnp.where(sid == NS - 1, Z_LAST // ZROWS, STRIPE // ZROWS)
    def _zero(k, c):
      zo = pl.multiple_of(zstart + k * ZROWS, 8)
      pltpu.sync_copy(zbuf, acc.at[pl.ds(zo, ZROWS)])
      return c
    lax.fori_loop(0, nz, _zero, 0)
    plsc.subcore_barrier()

    base0 = sid * TROWS
    pltpu.async_copy(im_hbm.at[pl.ds(base0, SUP)], ist.at[0], sem_st)
    pltpu.async_copy(ht_hbm.at[pl.ds(base0, SUP)], hst.at[0], sem_st)
    pltpu.async_copy(w_hbm.at[pl.ds(base0, SUP)], wstg.at[0], sem_st)

    def _step(step, carry):
      cm, p, nf = carry
      ssl = step & 1
      pltpu.make_async_copy(im_hbm.at[pl.ds(base0, SUP)], ist.at[ssl],
                            sem_st).wait()
      pltpu.make_async_copy(ht_hbm.at[pl.ds(base0, SUP)], hst.at[ssl],
                            sem_st).wait()
      pltpu.make_async_copy(w_hbm.at[pl.ds(base0, SUP)], wstg.at[ssl],
                            sem_st).wait()
      @pl.when(step + 1 < STEPS)
      def _():
        nxt = pl.multiple_of(base0 + (step + 1) * SUP, 8)
        pltpu.async_copy(im_hbm.at[pl.ds(nxt, SUP)], ist.at[1 - ssl],
                         sem_st)
        pltpu.async_copy(ht_hbm.at[pl.ds(nxt, SUP)], hst.at[1 - ssl],
                         sem_st)
        pltpu.async_copy(w_hbm.at[pl.ds(nxt, SUP)], wstg.at[1 - ssl],
                         sem_st)

      def _row(j, rc):
        cmr, pr, nfr = rc

        ims, lts, wvs, ms, cnts = [], [], [], [], []
        for k in range(BLK // L):
          s16 = pl.ds(k * L, L)
          ims.append(ist[ssl, j, s16])
          lt = hst[ssl, j, s16] - lo
          lts.append(lt)
          wvs.append(wstg[ssl, j, s16])
          m = (lt >= 0) & (lt < CHUNK)
          ms.append(m)
          cnts.append(jnp.sum(jnp.where(m, jnp.int32(1), jnp.int32(0))))
        base = cmr
        for k in range(BLK // L):
          plsc.store_compressed(r_im.at[pl.ds(base, L)], ims[k], mask=ms[k])
          plsc.store_compressed(r_ht.at[pl.ds(base, L)], lts[k], mask=ms[k])
          plsc.store_compressed(r_w.at[pl.ds(base, L)], wvs[k], mask=ms[k])
          base = base + cnts[k]
        @pl.when(base >= RING)
        def _():
          for i in range(BLK // L):
            s16 = pl.ds(RING + i * L, L)
            d16 = pl.ds(i * L, L)
            r_im[d16] = r_im[s16]
            r_ht[d16] = r_ht[s16]
            r_w[d16] = r_w[s16]
        pr = pr + (base - cmr)
        cmr = jnp.where(base >= RING, base - RING, base)

        do = pr >= BLK
        @pl.when(do)
        def _():
          for n in range(3):
            for s in range(2):
              @pl.when((nfr % 3 == n) & ((nfr & 1) == s))
              def _():
                _issue_nb(n, s, nfr)
          @pl.when(nfr >= 1)
          def _():
            for s in range(2):
              @pl.when(((nfr - 1) & 1) == s)
              def _():
                _drain_nb(s)
        pr = jnp.where(do, pr - BLK, pr)
        nfr = jnp.where(do, nfr + 1, nfr)
        return cmr, pr, nfr

      return lax.fori_loop(0, SUP, _row, (cm, p, nf))

    def _issue_nb(n, s, nf):
      @pl.when(nf >= 2)
      def _():
        pltpu.make_async_copy(rows2.at[s], acc.at[sht2.at[s]],
                              sem_s0 if s == 0 else sem_s1).wait()
      off = n * BLK
      for k in range(BLK // L):
        s16 = pl.ds(off + k * L, L)
        d16 = pl.ds(k * L, L)
        sht2[s, d16] = r_ht[s16]
        swt2[s, d16] = r_w[s16]
      pltpu.async_copy(x_hbm.at[r_im.at[pl.ds(off, BLK)]], rows2.at[s],
                       sem_g0 if s == 0 else sem_g1)

    def _drain_nb(s):
      pltpu.make_async_copy(x_hbm.at[r_im.at[pl.ds(0, BLK)]], rows2.at[s],
                            sem_g0 if s == 0 else sem_g1).wait()
      def _mul(k, c):
        wv16 = swt2[s, pl.ds(k * L, L)]
        for r in range(L):
          i = k * L + r
          wv = jnp.full((L,), wv16[r], jnp.float32)
          rows2[s, i, pl.ds(0, L)] = rows2[s, i, pl.ds(0, L)] * wv
          rows2[s, i, pl.ds(L, L)] = rows2[s, i, pl.ds(L, L)] * wv
        return c
      lax.fori_loop(0, BLK // L, _mul, 0, unroll=2)
      pltpu.async_copy(rows2.at[s], acc.at[sht2.at[s]],
                       sem_s0 if s == 0 else sem_s1, add=True)

    cm, p, nf = lax.fori_loop(
        0, STEPS, _step,
        (jnp.int32(0), jnp.int32(0), jnp.int32(0)))

    @pl.when(nf >= 1)
    def _():
      for s in range(2):
        @pl.when(((nf - 1) & 1) == s)
        def _():
          _drain_nb(s)

    for n in range(3):
      for s in range(2):
        @pl.when((nf % 3 == n) & ((nf & 1) == s))
        def _():
          off = n * BLK
          for k in range(BLK // L):
            s16 = pl.ds(off + k * L, L)
            keep = (lax.iota(jnp.int32, L) + (k * L)) < p
            r_ht[s16] = jnp.where(keep, r_ht[s16], CHUNK)
            r_w[s16] = jnp.where(keep, r_w[s16], 0.0)
          _issue_nb(n, s, nf)
          _drain_nb(s)

    for s_ in range(2):
      @pl.when((nf >= 1) & (((nf - 1) & 1) == s_))
      def _():
        pltpu.make_async_copy(rows2.at[s_], acc.at[sht2.at[s_]],
                              sem_s0 if s_ == 0 else sem_s1).wait()
      @pl.when((nf & 1) == s_)
      def _():
        pltpu.make_async_copy(rows2.at[s_], acc.at[sht2.at[s_]],
                              sem_s0 if s_ == 0 else sem_s1).wait()

    plsc.subcore_barrier()
    wb = pl.multiple_of(sid * STRIPE, 8)
    ob = pl.multiple_of(lo + sid * STRIPE, 8)
    @pl.when(sid < NS - 1)
    def _():
      pltpu.sync_copy(acc.at[pl.ds(wb, STRIPE)], out_hbm.at[pl.ds(ob, STRIPE)])
    @pl.when(sid == NS - 1)
    def _():
      pltpu.sync_copy(acc.at[pl.ds(wb, WB_LAST)],
                      out_hbm.at[pl.ds(ob, WB_LAST)])
    plsc.subcore_barrier()


@jax.jit
def _im2ht_sc(x, im2, ht2, w2):
  mesh = plsc.VectorSubcoreMesh(core_axis_name="c", subcore_axis_name="s")
  return pl.kernel(
      _sc_body,
      out_type=jax.ShapeDtypeStruct((N_BINS, C), jnp.float32),
      mesh=mesh,
      compiler_params=pltpu.CompilerParams(
          use_tc_tiling_on_sc=False, needs_layout_passes=False),
      scratch_types=[
          pltpu.VMEM_SHARED((ACC_ROWS, C), jnp.float32),
          pltpu.VMEM((2, SUP, BLK), jnp.int32),
          pltpu.VMEM((2, SUP, BLK), jnp.int32),
          pltpu.VMEM((2, SUP, BLK), jnp.float32),
          pltpu.VMEM((RSLACK,), jnp.int32),
          pltpu.VMEM((RSLACK,), jnp.int32),
          pltpu.VMEM((RSLACK,), jnp.float32),
          pltpu.VMEM((2, BLK), jnp.int32),
          pltpu.VMEM((2, BLK), jnp.float32),
          pltpu.VMEM((2, BLK, C), jnp.float32),
          pltpu.VMEM((ZROWS, C), jnp.float32),
          pltpu.SemaphoreType.DMA,
          pltpu.SemaphoreType.DMA,
          pltpu.SemaphoreType.DMA,
          pltpu.SemaphoreType.DMA,
          pltpu.SemaphoreType.DMA,
      ],
  )(x, im2, ht2, w2)


def kernel(input, vote_weight, vote_im_idx, vote_ht_idx):
  x = input.reshape(C, H * W).T
  pad = PADN - N_VOTES
  w2 = jnp.concatenate(
      [vote_weight, jnp.zeros((pad,), jnp.float32)]).reshape(VROWS, BLK)
  im2 = jnp.concatenate(
      [vote_im_idx.astype(jnp.int32),
       jnp.zeros((pad,), jnp.int32)]).reshape(VROWS, BLK)
  ht2 = jnp.concatenate(
      [vote_ht_idx.astype(jnp.int32),
       jnp.full((pad,), N_BINS, jnp.int32)]).reshape(VROWS, BLK)
  out = _im2ht_sc(x, im2, ht2, w2)
  return out.T.reshape(B, C, HT_H, HT_W)


# speedup vs baseline: 1.6309x; 1.6309x over previous
import jax
import jax.numpy as jnp
from jax import lax
from jax.experimental import pallas as pl
from jax.experimental.pallas import tpu as pltpu
from jax.experimental.pallas import tpu_sc as plsc

B, C, H, W = 1, 32, 512, 512
HT_H, HT_W = 400, 400
N_BINS = HT_H * HT_W
N_VOTES = 2000000
NC, NS, L = 2, 16, 16
N_CHUNKS = 4
CHUNK = N_BINS // N_CHUNKS
ACC_ROWS = CHUNK + 160
BLK = 128
PADN = 1 << 21
VROWS = PADN // BLK
TROWS = VROWS // NS
SUP = 16
STEPS = TROWS // SUP
RING = 3 * BLK
RSLACK = RING + BLK
STRIPE = 2560
WB_LAST = CHUNK - (NS - 1) * STRIPE
Z_LAST = ACC_ROWS - (NS - 1) * STRIPE
ZROWS = 160


def _sc_body(x_hbm, im_hbm, ht_hbm, w_hbm, out_hbm,
             acc, ist, hst, wstg, r_im, r_ht, r_w, sht2, swt2, rows2, zbuf,
             sem_st, sem_g0, sem_g1, sem_s0, sem_s1):
  cid = lax.axis_index("c")
  sid = lax.axis_index("s")

  def _zfill(i, c):
    zbuf[i, pl.ds(0, L)] = jnp.zeros((L,), jnp.float32)
    zbuf[i, pl.ds(L, L)] = jnp.zeros((L,), jnp.float32)
    return c
  lax.fori_loop(0, ZROWS, _zfill, 0)
  def _rfill(i, c):
    r_im[pl.ds(i * L, L)] = jnp.zeros((L,), jnp.int32)
    r_ht[pl.ds(i * L, L)] = jnp.full((L,), CHUNK, jnp.int32)
    r_w[pl.ds(i * L, L)] = jnp.zeros((L,), jnp.float32)
    return c
  lax.fori_loop(0, RSLACK // L, _rfill, 0)

  for chunk_i in range(N_CHUNKS // NC):
    lo = (cid * (N_CHUNKS // NC) + chunk_i) * CHUNK

    zstart = sid * STRIPE
    nz = jnp.where(sid == NS - 1, Z_LAST // ZROWS, STRIPE // ZROWS)
    def _zero(k, c):
      zo = pl.multiple_of(zstart + k * ZROWS, 8)
      pltpu.sync_copy(zbuf, acc.at[pl.ds(zo, ZROWS)])
      return c
    lax.fori_loop(0, nz, _zero, 0)
    plsc.subcore_barrier()

    base0 = sid * TROWS
    pltpu.async_copy(im_hbm.at[pl.ds(base0, SUP)], ist.at[0], sem_st)
    pltpu.async_copy(ht_hbm.at[pl.ds(base0, SUP)], hst.at[0], sem_st)
    pltpu.async_copy(w_hbm.at[pl.ds(base0, SUP)], wstg.at[0], sem_st)

    def _step(step, carry):
      cm, p, nf = carry
      ssl = step & 1
      pltpu.make_async_copy(im_hbm.at[pl.ds(base0, SUP)], ist.at[ssl],
                            sem_st).wait()
      pltpu.make_async_copy(ht_hbm.at[pl.ds(base0, SUP)], hst.at[ssl],
                            sem_st).wait()
      pltpu.make_async_copy(w_hbm.at[pl.ds(base0, SUP)], wstg.at[ssl],
                            sem_st).wait()
      @pl.when(step + 1 < STEPS)
      def _():
        nxt = pl.multiple_of(base0 + (step + 1) * SUP, 8)
        pltpu.async_copy(im_hbm.at[pl.ds(nxt, SUP)], ist.at[1 - ssl],
                         sem_st)
        pltpu.async_copy(ht_hbm.at[pl.ds(nxt, SUP)], hst.at[1 - ssl],
                         sem_st)
        pltpu.async_copy(w_hbm.at[pl.ds(nxt, SUP)], wstg.at[1 - ssl],
                         sem_st)

      def _row(j, rc):
        cmr, pr, nfr = rc

        ims, lts, wvs, ms, cnts = [], [], [], [], []
        for k in range(BLK // L):
          s16 = pl.ds(k * L, L)
          ims.append(ist[ssl, j, s16])
          lt = hst[ssl, j, s16] - lo
          lts.append(lt)
          wvs.append(wstg[ssl, j, s16])
          m = (lt >= 0) & (lt < CHUNK)
          ms.append(m)
          cnts.append(jnp.sum(jnp.where(m, jnp.int32(1), jnp.int32(0))))
        base = cmr
        for k in range(BLK // L):
          plsc.store_compressed(r_im.at[pl.ds(base, L)], ims[k], mask=ms[k])
          plsc.store_compressed(r_ht.at[pl.ds(base, L)], lts[k], mask=ms[k])
          plsc.store_compressed(r_w.at[pl.ds(base, L)], wvs[k], mask=ms[k])
          base = base + cnts[k]
        @pl.when(base >= RING)
        def _():
          for i in range(BLK // L):
            s16 = pl.ds(RING + i * L, L)
            d16 = pl.ds(i * L, L)
            r_im[d16] = r_im[s16]
            r_ht[d16] = r_ht[s16]
            r_w[d16] = r_w[s16]
        pr = pr + (base - cmr)
        cmr = jnp.where(base >= RING, base - RING, base)

        do = pr >= BLK
        @pl.when(do)
        def _():
          for n in range(3):
            for s in range(2):
              @pl.when((nfr % 3 == n) & ((nfr & 1) == s))
              def _():
                _issue_nb(n, s, nfr)
          @pl.when(nfr >= 1)
          def _():
            for s in range(2):
              @pl.when(((nfr - 1) & 1) == s)
              def _():
                _drain_nb(s)
        pr = jnp.where(do, pr - BLK, pr)
        nfr = jnp.where(do, nfr + 1, nfr)
        return cmr, pr, nfr

      return lax.fori_loop(0, SUP, _row, (cm, p, nf))

    def _issue_nb(n, s, nf):
      @pl.when(nf >= 2)
      def _():
        pltpu.make_async_copy(rows2.at[s], acc.at[sht2.at[s]],
                              sem_s0 if s == 0 else sem_s1).wait()
      off = n * BLK
      for k in range(BLK // L):
        s16 = pl.ds(off + k * L, L)
        d16 = pl.ds(k * L, L)
        sht2[s, d16] = r_ht[s16]
        swt2[s, d16] = r_w[s16]
      pltpu.async_copy(x_hbm.at[r_im.at[pl.ds(off, BLK)]], rows2.at[s],
                       sem_g0 if s == 0 else sem_g1)

    def _drain_nb(s):
      pltpu.make_async_copy(x_hbm.at[r_im.at[pl.ds(0, BLK)]], rows2.at[s],
                            sem_g0 if s == 0 else sem_g1).wait()
      def _mul(k, c):
        wv16 = swt2[s, pl.ds(k * L, L)]
        for r in range(L):
          i = k * L + r
          wv = jnp.full((L,), wv16[r], jnp.float32)
          rows2[s, i, pl.ds(0, L)] = rows2[s, i, pl.ds(0, L)] * wv
          rows2[s, i, pl.ds(L, L)] = rows2[s, i, pl.ds(L, L)] * wv
        return c
      lax.fori_loop(0, BLK // L, _mul, 0)
      pltpu.async_copy(rows2.at[s], acc.at[sht2.at[s]],
                       sem_s0 if s == 0 else sem_s1, add=True)

    cm, p, nf = lax.fori_loop(
        0, STEPS, _step,
        (jnp.int32(0), jnp.int32(0), jnp.int32(0)))

    @pl.when(nf >= 1)
    def _():
      for s in range(2):
        @pl.when(((nf - 1) & 1) == s)
        def _():
          _drain_nb(s)

    for n in range(3):
      for s in range(2):
        @pl.when((nf % 3 == n) & ((nf & 1) == s))
        def _():
          off = n * BLK
          for k in range(BLK // L):
            s16 = pl.ds(off + k * L, L)
            keep = (lax.iota(jnp.int32, L) + (k * L)) < p
            r_ht[s16] = jnp.where(keep, r_ht[s16], CHUNK)
            r_w[s16] = jnp.where(keep, r_w[s16], 0.0)
          _issue_nb(n, s, nf)
          _drain_nb(s)

    for s_ in range(2):
      @pl.when((nf >= 1) & (((nf - 1) & 1) == s_))
      def _():
        pltpu.make_async_copy(rows2.at[s_], acc.at[sht2.at[s_]],
                              sem_s0 if s_ == 0 else sem_s1).wait()
      @pl.when((nf & 1) == s_)
      def _():
        pltpu.make_async_copy(rows2.at[s_], acc.at[sht2.at[s_]],
                              sem_s0 if s_ == 0 else sem_s1).wait()

    plsc.subcore_barrier()
    wb = pl.multiple_of(sid * STRIPE, 8)
    ob = pl.multiple_of(lo + sid * STRIPE, 8)
    @pl.when(sid < NS - 1)
    def _():
      pltpu.sync_copy(acc.at[pl.ds(wb, STRIPE)], out_hbm.at[pl.ds(ob, STRIPE)])
    @pl.when(sid == NS - 1)
    def _():
      pltpu.sync_copy(acc.at[pl.ds(wb, WB_LAST)],
                      out_hbm.at[pl.ds(ob, WB_LAST)])
    plsc.subcore_barrier()


@jax.jit
def _im2ht_sc(x, im2, ht2, w2):
  mesh = plsc.VectorSubcoreMesh(core_axis_name="c", subcore_axis_name="s")
  return pl.kernel(
      _sc_body,
      out_type=jax.ShapeDtypeStruct((N_BINS, C), jnp.float32),
      mesh=mesh,
      compiler_params=pltpu.CompilerParams(
          use_tc_tiling_on_sc=False, needs_layout_passes=False),
      scratch_types=[
          pltpu.VMEM_SHARED((ACC_ROWS, C), jnp.float32),
          pltpu.VMEM((2, SUP, BLK), jnp.int32),
          pltpu.VMEM((2, SUP, BLK), jnp.int32),
          pltpu.VMEM((2, SUP, BLK), jnp.float32),
          pltpu.VMEM((RSLACK,), jnp.int32),
          pltpu.VMEM((RSLACK,), jnp.int32),
          pltpu.VMEM((RSLACK,), jnp.float32),
          pltpu.VMEM((2, BLK), jnp.int32),
          pltpu.VMEM((2, BLK), jnp.float32),
          pltpu.VMEM((2, BLK, C), jnp.float32),
          pltpu.VMEM((ZROWS, C), jnp.float32),
          pltpu.SemaphoreType.DMA,
          pltpu.SemaphoreType.DMA,
          pltpu.SemaphoreType.DMA,
          pltpu.SemaphoreType.DMA,
          pltpu.SemaphoreType.DMA,
      ],
  )(x, im2, ht2, w2)


def kernel(input, vote_weight, vote_im_idx, vote_ht_idx):
  x = input.reshape(C, H * W).T
  pad = PADN - N_VOTES
  w2 = jnp.concatenate(
      [vote_weight, jnp.zeros((pad,), jnp.float32)]).reshape(VROWS, BLK)
  im2 = jnp.concatenate(
      [vote_im_idx.astype(jnp.int32),
       jnp.zeros((pad,), jnp.int32)]).reshape(VROWS, BLK)
  ht2 = jnp.concatenate(
      [vote_ht_idx.astype(jnp.int32),
       jnp.full((pad,), N_BINS, jnp.int32)]).reshape(VROWS, BLK)
  out = _im2ht_sc(x, im2, ht2, w2)
  return out.T.reshape(B, C, HT_H, HT_W)
